# Initial kernel scaffold; baseline (speedup 1.0000x reference)
import jax
import jax.numpy as jnp
from jax.experimental import pallas as pl


def kernel(x, pos_edge_index, neg_edge_index, W1, b1, W2, b2, Wl1, bl1, Wl2, bl2):
    raise NotImplementedError("write your pallas kernel here")


# speedup vs baseline: 3.0932x; 3.0932x over previous
import functools

import jax
import jax.numpy as jnp
from jax import lax
from jax.experimental import pallas as pl
from jax.experimental.pallas import tpu as pltpu
from jax.experimental.pallas import tpu_sc as plsc

N = 10000
D = 128
H = 16
EP = 320000
ET = 640000

NC = 2
NS = 16
NW = NC * NS
L = 16

CHUNK = 128
SCHUNKS = 79
DCHUNKS = 157
ACC_ROWS = NS * 5 * CHUNK
JUNK_ROW = N

_mesh = plsc.VectorSubcoreMesh(core_axis_name="c", subcore_axis_name="s")
_sc_params = pltpu.CompilerParams(needs_layout_passes=False)


def _blocked_idx(idx, chunks, fill):
    total = NW * chunks * CHUNK
    pad = jnp.full((total - idx.shape[0],), fill, jnp.int32)
    return jnp.concatenate([idx, pad]).reshape(NW, chunks, CHUNK)


def _worker_id():
    return lax.axis_index("s") * NC + lax.axis_index("c")


def _fill_rows(ref, rows, width, value):
    v = jnp.full((L,), value, jnp.float32)

    def body(i, _):
        for k in range(width // L):
            ref[i, pl.ds(k * L, L)] = v
        return 0

    lax.fori_loop(0, rows, body, 0)


def _zero_acc(buf_v, acc_sh, sid):
    for k in range(5):
        pltpu.sync_copy(buf_v, acc_sh.at[pl.ds((sid * 5 + k) * CHUNK, CHUNK)])


def _read_out(acc_sh, out_hbm, cid, sid):
    rows = ACC_ROWS // NS
    pltpu.sync_copy(acc_sh.at[pl.ds(sid * rows, rows)],
                    out_hbm.at[cid, pl.ds(sid * rows, rows)])


@functools.partial(
    pl.kernel,
    out_type=jax.ShapeDtypeStruct((NC, ACC_ROWS, D), jnp.float32),
    mesh=_mesh,
    compiler_params=_sc_params,
    scratch_types=[
        pltpu.VMEM((SCHUNKS, CHUNK), jnp.int32),
        pltpu.VMEM((CHUNK, D), jnp.float32),
        pltpu.VMEM_SHARED((ACC_ROWS, D), jnp.float32),
    ],
)
def _deg_kernel(dst_hbm, out_hbm, idx_v, buf_v, acc_sh):
    cid = lax.axis_index("c")
    sid = lax.axis_index("s")
    wid = _worker_id()

    _fill_rows(buf_v, CHUNK, D, 0.0)
    _zero_acc(buf_v, acc_sh, sid)
    plsc.subcore_barrier()

    _fill_rows(buf_v, CHUNK, D, 1.0)
    pltpu.sync_copy(dst_hbm.at[wid], idx_v)

    def chunk(j, _):
        pltpu.sync_copy(buf_v, acc_sh.at[idx_v.at[j]], add=True)
        return 0

    lax.fori_loop(0, SCHUNKS, chunk, 0)
    plsc.subcore_barrier()
    _read_out(acc_sh, out_hbm, cid, sid)


def _make_scatter(width):
    @functools.partial(
        pl.kernel,
        out_type=jax.ShapeDtypeStruct((NC, ACC_ROWS, width), jnp.float32),
        mesh=_mesh,
        compiler_params=_sc_params,
        scratch_types=[
            pltpu.VMEM((SCHUNKS, CHUNK), jnp.int32),
            pltpu.VMEM((SCHUNKS, CHUNK), jnp.int32),
            pltpu.VMEM((CHUNK, width), jnp.float32),
            pltpu.VMEM_SHARED((ACC_ROWS, width), jnp.float32),
            pltpu.SemaphoreType.DMA,
        ],
    )
    def _scatter(u_hbm, src_hbm, dst_hbm, out_hbm, si_v, di_v, rows_v, acc_sh,
                 sem):
        cid = lax.axis_index("c")
        sid = lax.axis_index("s")
        wid = _worker_id()

        _fill_rows(rows_v, CHUNK, width, 0.0)
        _zero_acc(rows_v, acc_sh, sid)
        plsc.subcore_barrier()

        pltpu.sync_copy(src_hbm.at[wid], si_v)
        pltpu.sync_copy(dst_hbm.at[wid], di_v)

        def chunk(j, _):
            pltpu.async_copy(u_hbm.at[si_v.at[j]], rows_v, sem).wait()
            pltpu.sync_copy(rows_v, acc_sh.at[di_v.at[j]], add=True)
            return 0

        lax.fori_loop(0, SCHUNKS, chunk, 0)
        plsc.subcore_barrier()
        _read_out(acc_sh, out_hbm, cid, sid)

    return _scatter


_scatter_d = _make_scatter(D)


@functools.partial(
    pl.kernel,
    out_type=jax.ShapeDtypeStruct((NW * DCHUNKS * CHUNK,), jnp.float32),
    mesh=_mesh,
    compiler_params=_sc_params,
    scratch_types=[
        pltpu.VMEM((DCHUNKS, CHUNK), jnp.int32),
        pltpu.VMEM((DCHUNKS, CHUNK), jnp.int32),
        pltpu.VMEM((CHUNK, D), jnp.float32),
        pltpu.VMEM((CHUNK, D), jnp.float32),
        pltpu.VMEM((D,), jnp.float32),
        pltpu.VMEM((L,), jnp.float32),
        pltpu.VMEM((CHUNK,), jnp.float32),
        pltpu.SemaphoreType.DMA,
    ],
)
def _dec_kernel(hs_hbm, ht_hbm, src_hbm, tar_hbm, wl2_hbm, bl2_hbm, out_hbm,
                si_v, ti_v, bufs_v, buft_v, w_v, b_v, dot_v, sem):
    wid = _worker_id()

    pltpu.sync_copy(src_hbm.at[wid], si_v)
    pltpu.sync_copy(tar_hbm.at[wid], ti_v)
    pltpu.sync_copy(wl2_hbm, w_v)
    pltpu.sync_copy(bl2_hbm, b_v)

    wregs = [w_v[pl.ds(k * L, L)] for k in range(D // L)]
    bl2 = b_v[pl.ds(0, L)]
    zero = jnp.zeros((L,), jnp.float32)

    def chunk(j, _):
        pltpu.async_copy(hs_hbm.at[si_v.at[j]], bufs_v, sem).wait()
        pltpu.async_copy(ht_hbm.at[ti_v.at[j]], buft_v, sem).wait()

        def group(g, _):
            rows = g * L + lax.iota(jnp.int32, L)
            acc = zero
            for k in range(D // L):
                for d2 in range(L):
                    col = jnp.full((L,), k * L + d2, jnp.int32)
                    s = plsc.load_gather(bufs_v, [rows, col])
                    t = plsc.load_gather(buft_v, [rows, col])
                    acc = acc + jnp.maximum(s + t, 0.0) * wregs[k][d2]
            dot_v[pl.ds(g * L, L)] = 1.0 / (1.0 + jnp.exp(-(acc + bl2)))
            return 0

        lax.fori_loop(0, CHUNK // L, group, 0)
        pltpu.sync_copy(dot_v,
                        out_hbm.at[pl.ds(wid * (DCHUNKS * CHUNK) + j * CHUNK,
                                         CHUNK)])
        return 0

    lax.fori_loop(0, DCHUNKS, chunk, 0)


def _tc1_body(deg_ref, x_ref, dinv_ref, xd_ref):
    dp = deg_ref[...]
    deg = dp[0, :N, 0:1] + dp[1, :N, 0:1] + 1.0
    dinv = lax.rsqrt(deg)
    dinv_ref[...] = jnp.broadcast_to(dinv, (N, D))
    xd_ref[...] = x_ref[...] * dinv


def _tc2_body(acc1_ref, xd_ref, dinv_ref, w1_ref, b1_ref, w2_ref, u2_ref):
    a = acc1_ref[...]
    dinv = dinv_ref[...]
    t = a[0, :N] + a[1, :N] + xd_ref[...]
    tw = jnp.dot(t, w1_ref[...], preferred_element_type=jnp.float32)
    h1 = jnp.maximum(tw * dinv[:, 0:H] + b1_ref[...], 0.0)
    hw = jnp.dot(h1, w2_ref[...], preferred_element_type=jnp.float32)
    u2_ref[...] = hw * dinv


def _tc3_body(acc2_ref, u2_ref, dinv_ref, b2_ref, wl1_ref, bl1_ref,
              hs_ref, ht_ref):
    a = acc2_ref[...]
    h = (a[0, :N] + a[1, :N] + u2_ref[...]) * dinv_ref[...] + b2_ref[...]
    wl1 = wl1_ref[...]
    hs_ref[...] = jnp.dot(h, wl1[0:D], preferred_element_type=jnp.float32) \
        + bl1_ref[...]
    ht_ref[...] = jnp.dot(h, wl1[D:2 * D], preferred_element_type=jnp.float32)


_tc1 = pl.pallas_call(
    _tc1_body,
    out_shape=(jax.ShapeDtypeStruct((N, D), jnp.float32),
               jax.ShapeDtypeStruct((N, D), jnp.float32)))
_tc2 = pl.pallas_call(
    _tc2_body,
    out_shape=jax.ShapeDtypeStruct((N, D), jnp.float32))
_tc3 = pl.pallas_call(
    _tc3_body,
    out_shape=(jax.ShapeDtypeStruct((N, D), jnp.float32),
               jax.ShapeDtypeStruct((N, D), jnp.float32)))


def kernel(x, pos_edge_index, neg_edge_index, W1, b1, W2, b2, Wl1, bl1, Wl2,
           bl2):
    ps, pd = pos_edge_index[0], pos_edge_index[1]

    pd_blk = _blocked_idx(pd, SCHUNKS, JUNK_ROW)
    ps_blk = _blocked_idx(ps, SCHUNKS, 0)

    deg_parts = _deg_kernel(pd_blk)
    dinv, xd = _tc1(deg_parts, x)

    acc1 = _scatter_d(xd, ps_blk, pd_blk)
    u2 = _tc2(acc1, xd, dinv, W1, b1.reshape(1, H), W2)

    acc2 = _scatter_d(u2, ps_blk, pd_blk)
    hs, ht = _tc3(acc2, u2, dinv, b2.reshape(1, D), Wl1, bl1.reshape(1, D))

    src = _blocked_idx(jnp.concatenate([ps, neg_edge_index[0]]), DCHUNKS, 0)
    tar = _blocked_idx(jnp.concatenate([pd, neg_edge_index[1]]), DCHUNKS, 0)
    dec = _dec_kernel(hs, ht, src, tar, Wl2.reshape(D),
                      jnp.broadcast_to(bl2, (L,)))
    return dec[:ET].reshape(ET, 1)


# speedup vs baseline: 3.4929x; 1.1292x over previous
import functools

import jax
import jax.numpy as jnp
from jax import lax
from jax.experimental import pallas as pl
from jax.experimental.pallas import tpu as pltpu
from jax.experimental.pallas import tpu_sc as plsc

N = 10000
D = 128
H = 16
EP = 320000
ET = 640000

NC = 2
NS = 16
NW = NC * NS
L = 16

CHUNK = 128
SCHUNKS = 79
DCHUNKS = 160
NBUF = 2
ACC_ROWS = NS * 5 * CHUNK
JUNK_ROW = N

_mesh = plsc.VectorSubcoreMesh(core_axis_name="c", subcore_axis_name="s")
_sc_params = pltpu.CompilerParams(needs_layout_passes=False)


def _blocked_idx(idx, chunks, fill):
    total = NW * chunks * CHUNK
    pad = jnp.full((total - idx.shape[0],), fill, jnp.int32)
    return jnp.concatenate([idx, pad]).reshape(NW, chunks, CHUNK)


def _worker_id():
    return lax.axis_index("s") * NC + lax.axis_index("c")


def _fill_rows(ref, rows, width, value):
    v = jnp.full((L,), value, jnp.float32)

    def body(i, _):
        for k in range(width // L):
            ref[i, pl.ds(k * L, L)] = v
        return 0

    lax.fori_loop(0, rows, body, 0)


def _zero_acc(buf_v, acc_sh, sid):
    for k in range(5):
        pltpu.sync_copy(buf_v, acc_sh.at[pl.ds((sid * 5 + k) * CHUNK, CHUNK)])


def _read_out(acc_sh, out_hbm, cid, sid):
    rows = ACC_ROWS // NS
    pltpu.sync_copy(acc_sh.at[pl.ds(sid * rows, rows)],
                    out_hbm.at[cid, pl.ds(sid * rows, rows)])


@functools.partial(
    pl.kernel,
    out_type=jax.ShapeDtypeStruct((NC, ACC_ROWS, D), jnp.float32),
    mesh=_mesh,
    compiler_params=_sc_params,
    scratch_types=[
        pltpu.VMEM((SCHUNKS, CHUNK), jnp.int32),
        pltpu.VMEM((CHUNK, D), jnp.float32),
        pltpu.VMEM_SHARED((ACC_ROWS, D), jnp.float32),
    ],
)
def _deg_kernel(dst_hbm, out_hbm, idx_v, buf_v, acc_sh):
    cid = lax.axis_index("c")
    sid = lax.axis_index("s")
    wid = _worker_id()

    _fill_rows(buf_v, CHUNK, D, 0.0)
    _zero_acc(buf_v, acc_sh, sid)
    plsc.subcore_barrier()

    _fill_rows(buf_v, CHUNK, D, 1.0)
    pltpu.sync_copy(dst_hbm.at[wid], idx_v)

    def chunk(j, _):
        pltpu.sync_copy(buf_v, acc_sh.at[idx_v.at[j]], add=True)
        return 0

    lax.fori_loop(0, SCHUNKS, chunk, 0)
    plsc.subcore_barrier()
    _read_out(acc_sh, out_hbm, cid, sid)


def _make_scatter(width):
    @functools.partial(
        pl.kernel,
        out_type=jax.ShapeDtypeStruct((NC, ACC_ROWS, width), jnp.float32),
        mesh=_mesh,
        compiler_params=_sc_params,
        scratch_types=[
            pltpu.VMEM((SCHUNKS, CHUNK), jnp.int32),
            pltpu.VMEM((SCHUNKS, CHUNK), jnp.int32),
            pltpu.VMEM((CHUNK, width), jnp.float32),
            pltpu.VMEM_SHARED((ACC_ROWS, width), jnp.float32),
            pltpu.SemaphoreType.DMA,
        ],
    )
    def _scatter(u_hbm, src_hbm, dst_hbm, out_hbm, si_v, di_v, rows_v, acc_sh,
                 sem):
        cid = lax.axis_index("c")
        sid = lax.axis_index("s")
        wid = _worker_id()

        _fill_rows(rows_v, CHUNK, width, 0.0)
        _zero_acc(rows_v, acc_sh, sid)
        plsc.subcore_barrier()

        pltpu.sync_copy(src_hbm.at[wid], si_v)
        pltpu.sync_copy(dst_hbm.at[wid], di_v)

        def chunk(j, _):
            pltpu.async_copy(u_hbm.at[si_v.at[j]], rows_v, sem).wait()
            pltpu.sync_copy(rows_v, acc_sh.at[di_v.at[j]], add=True)
            return 0

        lax.fori_loop(0, SCHUNKS, chunk, 0)
        plsc.subcore_barrier()
        _read_out(acc_sh, out_hbm, cid, sid)

    return _scatter


_scatter_d = _make_scatter(D)


@functools.partial(
    pl.kernel,
    out_type=jax.ShapeDtypeStruct((NW * DCHUNKS * CHUNK,), jnp.float32),
    mesh=_mesh,
    compiler_params=_sc_params,
    scratch_types=[
        pltpu.VMEM((DCHUNKS, CHUNK), jnp.int32),
        pltpu.VMEM((DCHUNKS, CHUNK), jnp.int32),
        pltpu.VMEM((NBUF, CHUNK, D), jnp.float32),
        pltpu.VMEM((NBUF, CHUNK, D), jnp.float32),
        pltpu.VMEM((D,), jnp.float32),
        pltpu.VMEM((L,), jnp.float32),
        pltpu.VMEM((CHUNK,), jnp.float32),
        [pltpu.SemaphoreType.DMA] * NBUF,
    ],
)
def _dec_kernel(hs_hbm, ht_hbm, src_hbm, tar_hbm, wl2_hbm, bl2_hbm, out_hbm,
                si_v, ti_v, bufs_v, buft_v, w_v, b_v, dot_v, sems):
    wid = _worker_id()

    pltpu.sync_copy(src_hbm.at[wid], si_v)
    pltpu.sync_copy(tar_hbm.at[wid], ti_v)
    pltpu.sync_copy(wl2_hbm, w_v)
    pltpu.sync_copy(bl2_hbm, b_v)

    wregs = [w_v[pl.ds(k * L, L)] for k in range(D // L)]
    bl2 = b_v[pl.ds(0, L)]
    zero = jnp.zeros((L,), jnp.float32)

    def fire(j, b):
        pltpu.async_copy(hs_hbm.at[si_v.at[j]], bufs_v.at[b], sems[b])
        pltpu.async_copy(ht_hbm.at[ti_v.at[j]], buft_v.at[b], sems[b])

    def drain(j, b):
        pltpu.make_async_copy(hs_hbm.at[si_v.at[j]], bufs_v.at[b],
                              sems[b]).wait()
        pltpu.make_async_copy(ht_hbm.at[ti_v.at[j]], buft_v.at[b],
                              sems[b]).wait()

    for b in range(NBUF):
        fire(b, b)

    def outer(grp, _):
        for b in range(NBUF):
            j = grp * NBUF + b
            drain(j, b)

            def group(g, _):
                rows = g * L + lax.iota(jnp.int32, L)
                acc = zero
                for k in range(D // L):
                    for d2 in range(L):
                        col = jnp.full((L,), k * L + d2, jnp.int32)
                        s = plsc.load_gather(bufs_v, [jnp.full((L,), b), rows,
                                                      col])
                        t = plsc.load_gather(buft_v, [jnp.full((L,), b), rows,
                                                      col])
                        acc = acc + jnp.maximum(s + t, 0.0) * wregs[k][d2]
                dot_v[pl.ds(g * L, L)] = 1.0 / (1.0 + jnp.exp(-(acc + bl2)))
                return 0

            lax.fori_loop(0, CHUNK // L, group, 0)

            @pl.when(j + NBUF < DCHUNKS)
            def _():
                fire(j + NBUF, b)

            pltpu.sync_copy(dot_v,
                            out_hbm.at[pl.ds(wid * (DCHUNKS * CHUNK)
                                             + j * CHUNK, CHUNK)])
        return 0

    lax.fori_loop(0, DCHUNKS // NBUF, outer, 0)


def _tc1_body(deg_ref, x_ref, dinv_ref, xd_ref):
    dp = deg_ref[...]
    deg = dp[0, :N, 0:1] + dp[1, :N, 0:1] + 1.0
    dinv = lax.rsqrt(deg)
    dinv_ref[...] = jnp.broadcast_to(dinv, (N, D))
    xd_ref[...] = x_ref[...] * dinv


def _tc2_body(acc1_ref, xd_ref, dinv_ref, w1_ref, b1_ref, w2_ref, u2_ref):
    a = acc1_ref[...]
    dinv = dinv_ref[...]
    t = a[0, :N] + a[1, :N] + xd_ref[...]
    tw = jnp.dot(t, w1_ref[...], preferred_element_type=jnp.float32)
    h1 = jnp.maximum(tw * dinv[:, 0:H] + b1_ref[...], 0.0)
    hw = jnp.dot(h1, w2_ref[...], preferred_element_type=jnp.float32)
    u2_ref[...] = hw * dinv


def _tc3_body(acc2_ref, u2_ref, dinv_ref, b2_ref, wl1_ref, bl1_ref,
              hs_ref, ht_ref):
    a = acc2_ref[...]
    h = (a[0, :N] + a[1, :N] + u2_ref[...]) * dinv_ref[...] + b2_ref[...]
    wl1 = wl1_ref[...]
    hs_ref[...] = jnp.dot(h, wl1[0:D], preferred_element_type=jnp.float32) \
        + bl1_ref[...]
    ht_ref[...] = jnp.dot(h, wl1[D:2 * D], preferred_element_type=jnp.float32)


_tc1 = pl.pallas_call(
    _tc1_body,
    out_shape=(jax.ShapeDtypeStruct((N, D), jnp.float32),
               jax.ShapeDtypeStruct((N, D), jnp.float32)))
_tc2 = pl.pallas_call(
    _tc2_body,
    out_shape=jax.ShapeDtypeStruct((N, D), jnp.float32))
_tc3 = pl.pallas_call(
    _tc3_body,
    out_shape=(jax.ShapeDtypeStruct((N, D), jnp.float32),
               jax.ShapeDtypeStruct((N, D), jnp.float32)))


def kernel(x, pos_edge_index, neg_edge_index, W1, b1, W2, b2, Wl1, bl1, Wl2,
           bl2):
    ps, pd = pos_edge_index[0], pos_edge_index[1]

    pd_blk = _blocked_idx(pd, SCHUNKS, JUNK_ROW)
    ps_blk = _blocked_idx(ps, SCHUNKS, 0)

    deg_parts = _deg_kernel(pd_blk)
    dinv, xd = _tc1(deg_parts, x)

    acc1 = _scatter_d(xd, ps_blk, pd_blk)
    u2 = _tc2(acc1, xd, dinv, W1, b1.reshape(1, H), W2)

    acc2 = _scatter_d(u2, ps_blk, pd_blk)
    hs, ht = _tc3(acc2, u2, dinv, b2.reshape(1, D), Wl1, bl1.reshape(1, D))

    src = _blocked_idx(jnp.concatenate([ps, neg_edge_index[0]]), DCHUNKS, 0)
    tar = _blocked_idx(jnp.concatenate([pd, neg_edge_index[1]]), DCHUNKS, 0)
    dec = _dec_kernel(hs, ht, src, tar, Wl2.reshape(D),
                      jnp.broadcast_to(bl2, (L,)))
    return dec[:ET].reshape(ET, 1)


# speedup vs baseline: 7.5335x; 2.1568x over previous
import functools

import jax
import jax.numpy as jnp
from jax import lax
from jax.experimental import pallas as pl
from jax.experimental.pallas import tpu as pltpu
from jax.experimental.pallas import tpu_sc as plsc

N = 10000
D = 128
H = 16
EP = 320000
ET = 640000

NC = 2
NS = 16
NW = NC * NS
L = 16

CHUNK = 128
SCHUNKS = 79
DCHUNKS = 160
NBUF = 2
ACC_ROWS = NS * 5 * CHUNK
JUNK_ROW = N

_mesh = plsc.VectorSubcoreMesh(core_axis_name="c", subcore_axis_name="s")
_sc_params = pltpu.CompilerParams(needs_layout_passes=False)


def _blocked_idx(idx, chunks, fill):
    total = NW * chunks * CHUNK
    pad = jnp.full((total - idx.shape[0],), fill, jnp.int32)
    return jnp.concatenate([idx, pad]).reshape(NW, chunks, CHUNK)


def _worker_id():
    return lax.axis_index("s") * NC + lax.axis_index("c")


def _fill_rows(ref, rows, width, value):
    v = jnp.full((L,), value, jnp.float32)

    def body(i, _):
        for k in range(width // L):
            ref[i, pl.ds(k * L, L)] = v
        return 0

    lax.fori_loop(0, rows, body, 0)


def _zero_acc(buf_v, acc_sh, sid):
    for k in range(5):
        pltpu.sync_copy(buf_v, acc_sh.at[pl.ds((sid * 5 + k) * CHUNK, CHUNK)])


def _read_out(acc_sh, out_hbm, cid, sid):
    rows = ACC_ROWS // NS
    pltpu.sync_copy(acc_sh.at[pl.ds(sid * rows, rows)],
                    out_hbm.at[cid, pl.ds(sid * rows, rows)])


@functools.partial(
    pl.kernel,
    out_type=jax.ShapeDtypeStruct((NC, ACC_ROWS, D), jnp.float32),
    mesh=_mesh,
    compiler_params=_sc_params,
    scratch_types=[
        pltpu.VMEM((SCHUNKS, CHUNK), jnp.int32),
        pltpu.VMEM((CHUNK, D), jnp.float32),
        pltpu.VMEM_SHARED((ACC_ROWS, D), jnp.float32),
    ],
)
def _deg_kernel(dst_hbm, out_hbm, idx_v, buf_v, acc_sh):
    cid = lax.axis_index("c")
    sid = lax.axis_index("s")
    wid = _worker_id()

    _fill_rows(buf_v, CHUNK, D, 0.0)
    _zero_acc(buf_v, acc_sh, sid)
    plsc.subcore_barrier()

    _fill_rows(buf_v, CHUNK, D, 1.0)
    pltpu.sync_copy(dst_hbm.at[wid], idx_v)

    def chunk(j, _):
        pltpu.sync_copy(buf_v, acc_sh.at[idx_v.at[j]], add=True)
        return 0

    lax.fori_loop(0, SCHUNKS, chunk, 0)
    plsc.subcore_barrier()
    _read_out(acc_sh, out_hbm, cid, sid)


def _make_scatter(width):
    @functools.partial(
        pl.kernel,
        out_type=jax.ShapeDtypeStruct((NC, ACC_ROWS, width), jnp.float32),
        mesh=_mesh,
        compiler_params=_sc_params,
        scratch_types=[
            pltpu.VMEM((SCHUNKS, CHUNK), jnp.int32),
            pltpu.VMEM((SCHUNKS, CHUNK), jnp.int32),
            pltpu.VMEM((CHUNK, width), jnp.float32),
            pltpu.VMEM_SHARED((ACC_ROWS, width), jnp.float32),
            pltpu.SemaphoreType.DMA,
        ],
    )
    def _scatter(u_hbm, src_hbm, dst_hbm, out_hbm, si_v, di_v, rows_v, acc_sh,
                 sem):
        cid = lax.axis_index("c")
        sid = lax.axis_index("s")
        wid = _worker_id()

        _fill_rows(rows_v, CHUNK, width, 0.0)
        _zero_acc(rows_v, acc_sh, sid)
        plsc.subcore_barrier()

        pltpu.sync_copy(src_hbm.at[wid], si_v)
        pltpu.sync_copy(dst_hbm.at[wid], di_v)

        def chunk(j, _):
            pltpu.async_copy(u_hbm.at[si_v.at[j]], rows_v, sem).wait()
            pltpu.sync_copy(rows_v, acc_sh.at[di_v.at[j]], add=True)
            return 0

        lax.fori_loop(0, SCHUNKS, chunk, 0)
        plsc.subcore_barrier()
        _read_out(acc_sh, out_hbm, cid, sid)

    return _scatter


_scatter_d = _make_scatter(D)


@functools.partial(
    pl.kernel,
    out_type=jax.ShapeDtypeStruct((NW * DCHUNKS * CHUNK,), jnp.float32),
    mesh=_mesh,
    compiler_params=_sc_params,
    scratch_types=[
        pltpu.VMEM((DCHUNKS, CHUNK), jnp.int32),
        pltpu.VMEM((DCHUNKS, CHUNK), jnp.int32),
        pltpu.VMEM((NBUF, CHUNK, D), jnp.float32),
        pltpu.VMEM((NBUF, CHUNK, D), jnp.float32),
        pltpu.VMEM((D,), jnp.float32),
        pltpu.VMEM((L,), jnp.float32),
        pltpu.VMEM((CHUNK,), jnp.float32),
        pltpu.VMEM((CHUNK, 17), jnp.float32),
        [pltpu.SemaphoreType.DMA] * NBUF,
    ],
)
def _dec_kernel(hs_hbm, ht_hbm, src_hbm, tar_hbm, wl2_hbm, bl2_hbm, out_hbm,
                si_v, ti_v, bufs_v, buft_v, w_v, b_v, dot_v, r_v, sems):
    wid = _worker_id()

    pltpu.sync_copy(src_hbm.at[wid], si_v)
    pltpu.sync_copy(tar_hbm.at[wid], ti_v)
    pltpu.sync_copy(wl2_hbm, w_v)
    pltpu.sync_copy(bl2_hbm, b_v)

    wregs = [w_v[pl.ds(k * L, L)] for k in range(D // L)]
    bl2 = b_v[pl.ds(0, L)]
    zero = jnp.zeros((L,), jnp.float32)

    def fire(j, b):
        pltpu.async_copy(hs_hbm.at[si_v.at[j]], bufs_v.at[b], sems[b])
        pltpu.async_copy(ht_hbm.at[ti_v.at[j]], buft_v.at[b], sems[b])

    def drain(j, b):
        pltpu.make_async_copy(hs_hbm.at[si_v.at[j]], bufs_v.at[b],
                              sems[b]).wait()
        pltpu.make_async_copy(ht_hbm.at[ti_v.at[j]], buft_v.at[b],
                              sems[b]).wait()

    for b in range(NBUF):
        fire(b, b)

    def outer(grp, _):
        for b in range(NBUF):
            j = grp * NBUF + b
            drain(j, b)

            def edge(e, _):
                racc = zero
                for k in range(D // L):
                    s = bufs_v[b, e, pl.ds(k * L, L)]
                    t = buft_v[b, e, pl.ds(k * L, L)]
                    racc = racc + jnp.maximum(s + t, 0.0) * wregs[k]
                r_v[e, pl.ds(0, L)] = racc
                return 0

            lax.fori_loop(0, CHUNK, edge, 0)

            def group(g, _):
                rows = g * L + lax.iota(jnp.int32, L)
                acc = zero
                for d2 in range(L):
                    col = jnp.full((L,), d2, jnp.int32)
                    acc = acc + plsc.load_gather(r_v, [rows, col])
                dot_v[pl.ds(g * L, L)] = 1.0 / (1.0 + jnp.exp(-(acc + bl2)))
                return 0

            lax.fori_loop(0, CHUNK // L, group, 0)

            @pl.when(j + NBUF < DCHUNKS)
            def _():
                fire(j + NBUF, b)

            pltpu.sync_copy(dot_v,
                            out_hbm.at[pl.ds(wid * (DCHUNKS * CHUNK)
                                             + j * CHUNK, CHUNK)])
        return 0

    lax.fori_loop(0, DCHUNKS // NBUF, outer, 0)


def _tc1_body(deg_ref, x_ref, dinv_ref, xd_ref):
    dp = deg_ref[...]
    deg = dp[0, :N, 0:1] + dp[1, :N, 0:1] + 1.0
    dinv = lax.rsqrt(deg)
    dinv_ref[...] = jnp.broadcast_to(dinv, (N, D))
    xd_ref[...] = x_ref[...] * dinv


def _tc2_body(acc1_ref, xd_ref, dinv_ref, w1_ref, b1_ref, w2_ref, u2_ref):
    a = acc1_ref[...]
    dinv = dinv_ref[...]
    t = a[0, :N] + a[1, :N] + xd_ref[...]
    tw = jnp.dot(t, w1_ref[...], preferred_element_type=jnp.float32)
    h1 = jnp.maximum(tw * dinv[:, 0:H] + b1_ref[...], 0.0)
    hw = jnp.dot(h1, w2_ref[...], preferred_element_type=jnp.float32)
    u2_ref[...] = hw * dinv


def _tc3_body(acc2_ref, u2_ref, dinv_ref, b2_ref, wl1_ref, bl1_ref,
              hs_ref, ht_ref):
    a = acc2_ref[...]
    h = (a[0, :N] + a[1, :N] + u2_ref[...]) * dinv_ref[...] + b2_ref[...]
    wl1 = wl1_ref[...]
    hs_ref[...] = jnp.dot(h, wl1[0:D], preferred_element_type=jnp.float32) \
        + bl1_ref[...]
    ht_ref[...] = jnp.dot(h, wl1[D:2 * D], preferred_element_type=jnp.float32)


_tc1 = pl.pallas_call(
    _tc1_body,
    out_shape=(jax.ShapeDtypeStruct((N, D), jnp.float32),
               jax.ShapeDtypeStruct((N, D), jnp.float32)))
_tc2 = pl.pallas_call(
    _tc2_body,
    out_shape=jax.ShapeDtypeStruct((N, D), jnp.float32))
_tc3 = pl.pallas_call(
    _tc3_body,
    out_shape=(jax.ShapeDtypeStruct((N, D), jnp.float32),
               jax.ShapeDtypeStruct((N, D), jnp.float32)))


def kernel(x, pos_edge_index, neg_edge_index, W1, b1, W2, b2, Wl1, bl1, Wl2,
           bl2):
    ps, pd = pos_edge_index[0], pos_edge_index[1]

    pd_blk = _blocked_idx(pd, SCHUNKS, JUNK_ROW)
    ps_blk = _blocked_idx(ps, SCHUNKS, 0)

    deg_parts = _deg_kernel(pd_blk)
    dinv, xd = _tc1(deg_parts, x)

    acc1 = _scatter_d(xd, ps_blk, pd_blk)
    u2 = _tc2(acc1, xd, dinv, W1, b1.reshape(1, H), W2)

    acc2 = _scatter_d(u2, ps_blk, pd_blk)
    hs, ht = _tc3(acc2, u2, dinv, b2.reshape(1, D), Wl1, bl1.reshape(1, D))

    src = _blocked_idx(jnp.concatenate([ps, neg_edge_index[0]]), DCHUNKS, 0)
    tar = _blocked_idx(jnp.concatenate([pd, neg_edge_index[1]]), DCHUNKS, 0)
    dec = _dec_kernel(hs, ht, src, tar, Wl2.reshape(D),
                      jnp.broadcast_to(bl2, (L,)))
    return dec[:ET].reshape(ET, 1)
